# baseline (speedup 1.0000x reference)
import dataclasses
import math

import jax
import jax.numpy as jnp
import numpy as np
from jax import lax
from jax.experimental import pallas as pl
from jax.experimental.pallas import tpu as pltpu
from jax.experimental.pallas import tpu_sc as plsc

D = 64
VOCAB = 1000000
SEQ = 200
B = 1024
N = SEQ * B
NW = 32
PER_W = N // NW
CH = 128
NCH = PER_W // CH
NG = CH // 16
SCALE = 8.0

HOUR_OFF = 0
MIN_OFF = 24
WD_OFF = 28
MODE_OFF = 35
PE_OFF = 43
SV_ROWS = PE_OFF + SEQ


def _pos_encoding_np():
    den = np.exp(-np.arange(0, D, 2, dtype=np.float32) * (math.log(10000.0) / D))
    pos = np.arange(0, SEQ, dtype=np.float32).reshape(SEQ, 1)
    pe = np.zeros((SEQ, D), dtype=np.float32)
    pe[:, 0::2] = np.sin(pos * den)
    pe[:, 1::2] = np.cos(pos * den)
    return pe


_PE = _pos_encoding_np()


def _sc_kernel_body(idx_hbm, smalls_hbm, loc_hbm, out_hbm,
                    ib0, ib1, wm0, wm1, sr0, sr1, tb0, tb1, ob0, ob1,
                    rows0, rows1, att0, att1, atw0, atw1,
                    sv, tt, twm,
                    gsem0, gsem1, tsem0, tsem1, wsem0, wsem1, osem0, osem1):
    ib = (ib0, ib1)
    wmb = (wm0, wm1)
    srb = (sr0, sr1)
    ttb = (tb0, tb1)
    obi = (ob0, ob1)
    rows = (rows0, rows1)
    att = (att0, att1)
    atw = (atw0, atw1)
    gsem = (gsem0, gsem1)
    tsem = (tsem0, tsem1)
    wsem = (wsem0, wsem1)
    osem = (osem0, osem1)

    wid = lax.axis_index("subcore") * 2 + lax.axis_index("core")
    start = wid * PER_W

    pltpu.sync_copy(smalls_hbm, sv)

    @pl.when(lax.axis_index("subcore") == 0)
    def _():
        @pl.loop(0, 96)
        def _(t):
            h = t // 4
            m = t % 4
            for j in range(4):
                sl = pl.ds(j * 16, 16)
                att0[t, sl] = (sv[HOUR_OFF + h, sl] + sv[MIN_OFF + m, sl]) * SCALE

        @pl.loop(0, 56)
        def _(i):
            wd = i // 8
            mo = i % 8
            for j in range(4):
                sl = pl.ds(j * 16, 16)
                atw0[i, sl] = (sv[WD_OFF + wd, sl] + sv[MODE_OFF + mo, sl]) * SCALE

        pltpu.sync_copy(att0.at[pl.ds(0, 96)], tt)
        pltpu.sync_copy(atw0.at[pl.ds(0, 56)], twm)

    plsc.subcore_barrier()

    zero16 = jnp.zeros((16,), jnp.int32)
    iota16 = lax.broadcasted_iota(jnp.int32, (16,), 0)
    ones16 = zero16 + 1

    def fire_gathers(c, bi):
        base = start + c * CH
        pltpu.sync_copy(idx_hbm.at[:, pl.ds(base, CH)], ib[bi])

        evc = []
        for g in range(NG):
            par = ib[bi][0, pl.ds(g * 16, 16)] & 1
            evc.append(jnp.sum(ones16 - par))
        evbase = []
        odbase = []
        ev_run = 0
        od_run = 0
        for g in range(NG):
            evbase.append(ev_run)
            odbase.append(od_run)
            ev_run = ev_run + evc[g]
            od_run = od_run + (16 - evc[g])
        nev = ev_run

        for g in range(NG):
            gsl = pl.ds(g * 16, 16)
            s16 = ib[bi][0, gsl]
            par = s16 & 1
            npar = ones16 - par
            ev_x = jnp.cumsum(npar) - npar
            od_x = jnp.cumsum(par) - par
            dest = jnp.where(par == 0, evbase[g] + ev_x, nev + odbase[g] + od_x)
            plsc.store_scatter(srb[bi], [dest], s16 >> 1)
            plsc.store_scatter(ttb[bi], [dest], ib[bi][1, gsl])
            plsc.store_scatter(wmb[bi], [dest],
                               ib[bi][2, gsl] * 8 + ib[bi][3, gsl])
            plsc.store_scatter(obi[bi], [zero16, dest], base + g * 16 + iota16)

        pltpu.make_async_copy(loc_hbm.at[srb[bi]], rows[bi], gsem[bi]).start()
        pltpu.make_async_copy(tt.at[ttb[bi]], att[bi], tsem[bi]).start()
        pltpu.make_async_copy(twm.at[wmb[bi]], atw[bi], wsem[bi]).start()
        return nev

    def wait_gathers(bi):
        pltpu.make_async_copy(loc_hbm.at[srb[bi]], rows[bi], gsem[bi]).wait()
        pltpu.make_async_copy(tt.at[ttb[bi]], att[bi], tsem[bi]).wait()
        pltpu.make_async_copy(twm.at[wmb[bi]], atw[bi], wsem[bi]).wait()

    def fire_out(bi):
        pltpu.make_async_copy(att[bi], out_hbm.at[obi[bi].at[0]], osem[bi]).start()

    def wait_out(bi):
        pltpu.make_async_copy(att[bi], out_hbm.at[obi[bi].at[0]], osem[bi]).wait()

    def compute_chunk(c, bi, nev):
        s = (start + c * CH) // B
        pe_regs = [sv[PE_OFF + s, pl.ds(j * 16, 16)] for j in range(4)]
        rb = rows[bi]
        ab = att[bi]
        wb = atw[bi]

        @pl.loop(0, nev)
        def _(e):
            for j in range(4):
                sl = pl.ds(j * 16, 16)
                ab[e, sl] = rb[e, sl] * SCALE + ab[e, sl] + wb[e, sl] + pe_regs[j]

        @pl.loop(nev, CH)
        def _(e):
            for j in range(4):
                sl = pl.ds(j * 16, 16)
                ab[e, sl] = (rb[e, pl.ds(D + j * 16, 16)] * SCALE
                             + ab[e, sl] + wb[e, sl] + pe_regs[j])

    nev0 = fire_gathers(0, 0)

    def loop_body(i, nev_carry):
        nevs = [nev_carry, nev_carry]
        for b01 in (0, 1):
            c = i * 2 + b01
            nb = 1 - b01
            wait_gathers(b01)
            nev_cur = nevs[b01]

            nev_next = lax.cond(
                c + 1 < NCH,
                lambda: _fire_next(c, nb),
                lambda: jnp.int32(0),
            )
            nevs[nb] = nev_next

            compute_chunk(c, b01, nev_cur)
            fire_out(b01)
        return nevs[0]

    def _fire_next(c, nb):
        @pl.when(c >= 1)
        def _():
            wait_out(nb)

        return jnp.int32(fire_gathers(c + 1, nb))

    lax.fori_loop(0, NCH // 2, loop_body, jnp.int32(nev0))

    wait_out(0)
    wait_out(1)


def kernel(src, time, weekday, mode, emb_loc_w, emb_mode_w, minute_w, hour_w, weekday_w):
    idx_packed = jnp.stack([
        src.reshape(-1).astype(jnp.int32),
        time.reshape(-1).astype(jnp.int32),
        weekday.reshape(-1).astype(jnp.int32),
        mode.reshape(-1).astype(jnp.int32),
    ])
    smalls = jnp.concatenate(
        [hour_w, minute_w, weekday_w, emb_mode_w, jnp.asarray(_PE)], axis=0
    )
    loc_pairs = emb_loc_w.reshape(VOCAB // 2, 2 * D)

    mesh = plsc.VectorSubcoreMesh(core_axis_name="core", subcore_axis_name="subcore")

    cp = pltpu.CompilerParams(use_tc_tiling_on_sc=False)
    if "needs_layout_passes" in pltpu.CompilerParams.__dataclass_fields__:
        cp = dataclasses.replace(cp, needs_layout_passes=False)

    run = pl.kernel(
        _sc_kernel_body,
        out_type=jax.ShapeDtypeStruct((N, D), jnp.float32),
        mesh=mesh,
        compiler_params=cp,
        scratch_types=[
            pltpu.VMEM((4, CH), jnp.int32),
            pltpu.VMEM((4, CH), jnp.int32),
            pltpu.VMEM((CH,), jnp.int32),
            pltpu.VMEM((CH,), jnp.int32),
            pltpu.VMEM((CH,), jnp.int32),
            pltpu.VMEM((CH,), jnp.int32),
            pltpu.VMEM((CH,), jnp.int32),
            pltpu.VMEM((CH,), jnp.int32),
            pltpu.VMEM((1, CH), jnp.int32),
            pltpu.VMEM((1, CH), jnp.int32),
            pltpu.VMEM((CH, 2 * D), jnp.float32),
            pltpu.VMEM((CH, 2 * D), jnp.float32),
            pltpu.VMEM((CH, D), jnp.float32),
            pltpu.VMEM((CH, D), jnp.float32),
            pltpu.VMEM((CH, D), jnp.float32),
            pltpu.VMEM((CH, D), jnp.float32),
            pltpu.VMEM((SV_ROWS, D), jnp.float32),
            pltpu.VMEM_SHARED((96, D), jnp.float32),
            pltpu.VMEM_SHARED((56, D), jnp.float32),
        ] + [pltpu.SemaphoreType.DMA] * 8,
    )
    out = run(idx_packed, smalls, loc_pairs)
    return out.reshape(SEQ, B, D)


# speedup vs baseline: 1.2073x; 1.2073x over previous
import dataclasses
import math

import jax
import jax.numpy as jnp
import numpy as np
from jax import lax
from jax.experimental import pallas as pl
from jax.experimental.pallas import tpu as pltpu
from jax.experimental.pallas import tpu_sc as plsc

D = 64
VOCAB = 1000000
SEQ = 200
B = 1024
N = SEQ * B
NW = 32
PER_W = N // NW
CH = 128
NCH = PER_W // CH
SCALE = 8.0

HOUR_OFF = 0
MIN_OFF = 24
WD_OFF = 28
MODE_OFF = 35
PE_OFF = 43
SV_ROWS = PE_OFF + SEQ


def _pos_encoding_np():
    den = np.exp(-np.arange(0, D, 2, dtype=np.float32) * (math.log(10000.0) / D))
    pos = np.arange(0, SEQ, dtype=np.float32).reshape(SEQ, 1)
    pe = np.zeros((SEQ, D), dtype=np.float32)
    pe[:, 0::2] = np.sin(pos * den)
    pe[:, 1::2] = np.cos(pos * den)
    return pe


_PE = _pos_encoding_np()


def _sc_kernel_body(idx_hbm, smalls_hbm, loc_hbm, out_hbm,
                    ib0, ib1, wm0, wm1, rows0, rows1, att0, att1, atw0, atw1,
                    sv, tt, twm,
                    gsem0, gsem1, tsem0, tsem1, wsem0, wsem1, osem0, osem1):
    ib = (ib0, ib1)
    wmb = (wm0, wm1)
    rows = (rows0, rows1)
    att = (att0, att1)
    atw = (atw0, atw1)
    gsem = (gsem0, gsem1)
    tsem = (tsem0, tsem1)
    wsem = (wsem0, wsem1)
    osem = (osem0, osem1)

    wid = lax.axis_index("subcore") * 2 + lax.axis_index("core")
    start = wid * PER_W

    pltpu.sync_copy(smalls_hbm, sv)

    @pl.when(lax.axis_index("subcore") == 0)
    def _():
        @pl.loop(0, 96)
        def _(t):
            h = t // 4
            m = t % 4
            for j in range(4):
                sl = pl.ds(j * 16, 16)
                att0[t, sl] = (sv[HOUR_OFF + h, sl] + sv[MIN_OFF + m, sl]) * SCALE

        @pl.loop(0, 56)
        def _(i):
            wd = i // 8
            mo = i % 8
            for j in range(4):
                sl = pl.ds(j * 16, 16)
                atw0[i, sl] = (sv[WD_OFF + wd, sl] + sv[MODE_OFF + mo, sl]) * SCALE

        pltpu.sync_copy(att0.at[pl.ds(0, 96)], tt)
        pltpu.sync_copy(atw0.at[pl.ds(0, 56)], twm)

    plsc.subcore_barrier()

    def fire_gathers(c, bi):
        base = start + c * CH
        pltpu.sync_copy(idx_hbm.at[:, pl.ds(base, CH)], ib[bi])

        @pl.loop(0, CH // 16)
        def _(g):
            gsl = pl.ds(g * 16, 16)
            wmb[bi][gsl] = ib[bi][2, gsl] * 8 + ib[bi][3, gsl]

        pltpu.make_async_copy(loc_hbm.at[ib[bi].at[0]], rows[bi], gsem[bi]).start()
        pltpu.make_async_copy(tt.at[ib[bi].at[1]], att[bi], tsem[bi]).start()
        pltpu.make_async_copy(twm.at[wmb[bi]], atw[bi], wsem[bi]).start()

    def wait_gathers(bi):
        pltpu.make_async_copy(loc_hbm.at[ib[bi].at[0]], rows[bi], gsem[bi]).wait()
        pltpu.make_async_copy(tt.at[ib[bi].at[1]], att[bi], tsem[bi]).wait()
        pltpu.make_async_copy(twm.at[wmb[bi]], atw[bi], wsem[bi]).wait()

    def fire_out(c, bi):
        base = start + c * CH
        pltpu.make_async_copy(att[bi], out_hbm.at[pl.ds(base, CH)], osem[bi]).start()

    def wait_out(bi):
        pltpu.make_async_copy(att[bi], out_hbm.at[pl.ds(start, CH)], osem[bi]).wait()

    def compute_chunk(c, bi):
        s = (start + c * CH) // B
        pe_regs = [sv[PE_OFF + s, pl.ds(j * 16, 16)] for j in range(4)]
        rb = rows[bi]
        ab = att[bi]
        wb = atw[bi]

        @pl.loop(0, CH, step=2)
        def _(e0):
            for de in range(2):
                e = e0 + de
                for j in range(4):
                    sl = pl.ds(j * 16, 16)
                    ab[e, sl] = rb[e, sl] * SCALE + ab[e, sl] + wb[e, sl] + pe_regs[j]

    fire_gathers(0, 0)

    @pl.loop(0, NCH // 2)
    def _(i):
        for b01 in (0, 1):
            c = i * 2 + b01
            nb = 1 - b01
            wait_gathers(b01)

            @pl.when(c + 1 < NCH)
            def _():
                @pl.when(c >= 1)
                def _():
                    wait_out(nb)

                fire_gathers(c + 1, nb)

            compute_chunk(c, b01)
            fire_out(c, b01)

    wait_out(0)
    wait_out(1)


def kernel(src, time, weekday, mode, emb_loc_w, emb_mode_w, minute_w, hour_w, weekday_w):
    idx_packed = jnp.stack([
        src.reshape(-1).astype(jnp.int32),
        time.reshape(-1).astype(jnp.int32),
        weekday.reshape(-1).astype(jnp.int32),
        mode.reshape(-1).astype(jnp.int32),
    ])
    smalls = jnp.concatenate(
        [hour_w, minute_w, weekday_w, emb_mode_w, jnp.asarray(_PE)], axis=0
    )

    mesh = plsc.VectorSubcoreMesh(core_axis_name="core", subcore_axis_name="subcore")

    cp = pltpu.CompilerParams(use_tc_tiling_on_sc=False)
    if "needs_layout_passes" in pltpu.CompilerParams.__dataclass_fields__:
        cp = dataclasses.replace(cp, needs_layout_passes=False)

    run = pl.kernel(
        _sc_kernel_body,
        out_type=jax.ShapeDtypeStruct((N, D), jnp.float32),
        mesh=mesh,
        compiler_params=cp,
        scratch_types=[
            pltpu.VMEM((4, CH), jnp.int32),
            pltpu.VMEM((4, CH), jnp.int32),
            pltpu.VMEM((CH,), jnp.int32),
            pltpu.VMEM((CH,), jnp.int32),
            pltpu.VMEM((CH, D), jnp.float32),
            pltpu.VMEM((CH, D), jnp.float32),
            pltpu.VMEM((CH, D), jnp.float32),
            pltpu.VMEM((CH, D), jnp.float32),
            pltpu.VMEM((CH, D), jnp.float32),
            pltpu.VMEM((CH, D), jnp.float32),
            pltpu.VMEM((SV_ROWS, D), jnp.float32),
            pltpu.VMEM_SHARED((96, D), jnp.float32),
            pltpu.VMEM_SHARED((56, D), jnp.float32),
        ] + [pltpu.SemaphoreType.DMA] * 8,
    )
    out = run(idx_packed, smalls, emb_loc_w)
    return out.reshape(SEQ, B, D)


# speedup vs baseline: 1.2475x; 1.0334x over previous
import dataclasses
import math

import jax
import jax.numpy as jnp
import numpy as np
from jax import lax
from jax.experimental import pallas as pl
from jax.experimental.pallas import tpu as pltpu
from jax.experimental.pallas import tpu_sc as plsc

D = 64
VOCAB = 1000000
SEQ = 200
B = 1024
N = SEQ * B
NW = 32
PER_W = N // NW
CH = 128
NCH = PER_W // CH
SCALE = 8.0

HOUR_OFF = 0
MIN_OFF = 24
WD_OFF = 28
MODE_OFF = 35
PE_OFF = 43
SV_ROWS = PE_OFF + SEQ


def _pos_encoding_np():
    den = np.exp(-np.arange(0, D, 2, dtype=np.float32) * (math.log(10000.0) / D))
    pos = np.arange(0, SEQ, dtype=np.float32).reshape(SEQ, 1)
    pe = np.zeros((SEQ, D), dtype=np.float32)
    pe[:, 0::2] = np.sin(pos * den)
    pe[:, 1::2] = np.cos(pos * den)
    return pe


_PE = _pos_encoding_np()


def _sc_kernel_body(idx_hbm, smalls_hbm, loc_hbm, out_hbm,
                    iball, cidxall, rows0, rows1, att0, att1,
                    sv, t2,
                    gsem0, gsem1, tsem0, tsem1, osem0, osem1):
    rows = (rows0, rows1)
    att = (att0, att1)
    gsem = (gsem0, gsem1)
    tsem = (tsem0, tsem1)
    osem = (osem0, osem1)

    sid = lax.axis_index("subcore")
    wid = sid * 2 + lax.axis_index("core")
    start = wid * PER_W

    pltpu.sync_copy(smalls_hbm, sv)

    for tloc in range(6):
        t = sid * 6 + tloc
        h = t // 4
        m = t % 4
        ttrow = [sv[HOUR_OFF + h, pl.ds(j * 16, 16)]
                 + sv[MIN_OFF + m, pl.ds(j * 16, 16)] for j in range(4)]

        @pl.loop(0, 56)
        def _(i):
            wd = i // 8
            mo = i % 8
            for j in range(4):
                sl = pl.ds(j * 16, 16)
                att0[i, sl] = (ttrow[j] + sv[WD_OFF + wd, sl]
                               + sv[MODE_OFF + mo, sl]) * SCALE

        pltpu.sync_copy(att0.at[pl.ds(0, 56)], t2.at[pl.ds(t * 56, 56)])

    pltpu.sync_copy(idx_hbm.at[:, pl.ds(start, PER_W)], iball)

    @pl.loop(0, PER_W // 16)
    def _(g):
        gsl = pl.ds(g * 16, 16)
        cidxall[gsl] = (iball[1, gsl] * 56 + iball[2, gsl] * 8
                        + iball[3, gsl])

    plsc.subcore_barrier()

    def fire_gathers(c, bi):
        lb = c * CH
        pltpu.make_async_copy(
            loc_hbm.at[iball.at[0, pl.ds(lb, CH)]], rows[bi], gsem[bi]).start()
        pltpu.make_async_copy(
            t2.at[cidxall.at[pl.ds(lb, CH)]], att[bi], tsem[bi]).start()

    def wait_gathers(bi):
        pltpu.make_async_copy(
            loc_hbm.at[iball.at[0, pl.ds(0, CH)]], rows[bi], gsem[bi]).wait()
        pltpu.make_async_copy(
            t2.at[cidxall.at[pl.ds(0, CH)]], att[bi], tsem[bi]).wait()

    def fire_out(c, bi):
        base = start + c * CH
        pltpu.make_async_copy(att[bi], out_hbm.at[pl.ds(base, CH)], osem[bi]).start()

    def wait_out(bi):
        pltpu.make_async_copy(att[bi], out_hbm.at[pl.ds(start, CH)], osem[bi]).wait()

    def compute_chunk(c, bi):
        s = (start + c * CH) // B
        pe_regs = [sv[PE_OFF + s, pl.ds(j * 16, 16)] for j in range(4)]
        rb = rows[bi]
        ab = att[bi]

        @pl.loop(0, CH, step=2)
        def _(e0):
            for de in range(2):
                e = e0 + de
                for j in range(4):
                    sl = pl.ds(j * 16, 16)
                    ab[e, sl] = rb[e, sl] * SCALE + ab[e, sl] + pe_regs[j]

    fire_gathers(0, 0)

    @pl.loop(0, NCH // 2)
    def _(i):
        for b01 in (0, 1):
            c = i * 2 + b01
            nb = 1 - b01
            wait_gathers(b01)

            @pl.when(c + 1 < NCH)
            def _():
                @pl.when(c >= 1)
                def _():
                    wait_out(nb)

                fire_gathers(c + 1, nb)

            compute_chunk(c, b01)
            fire_out(c, b01)

    wait_out(0)
    wait_out(1)


def kernel(src, time, weekday, mode, emb_loc_w, emb_mode_w, minute_w, hour_w, weekday_w):
    idx_packed = jnp.stack([
        src.reshape(-1).astype(jnp.int32),
        time.reshape(-1).astype(jnp.int32),
        weekday.reshape(-1).astype(jnp.int32),
        mode.reshape(-1).astype(jnp.int32),
    ])
    smalls = jnp.concatenate(
        [hour_w, minute_w, weekday_w, emb_mode_w, jnp.asarray(_PE)], axis=0
    )

    mesh = plsc.VectorSubcoreMesh(core_axis_name="core", subcore_axis_name="subcore")

    cp = pltpu.CompilerParams(use_tc_tiling_on_sc=False)
    if "needs_layout_passes" in pltpu.CompilerParams.__dataclass_fields__:
        cp = dataclasses.replace(cp, needs_layout_passes=False)

    run = pl.kernel(
        _sc_kernel_body,
        out_type=jax.ShapeDtypeStruct((N, D), jnp.float32),
        mesh=mesh,
        compiler_params=cp,
        scratch_types=[
            pltpu.VMEM((4, PER_W), jnp.int32),
            pltpu.VMEM((PER_W,), jnp.int32),
            pltpu.VMEM((CH, D), jnp.float32),
            pltpu.VMEM((CH, D), jnp.float32),
            pltpu.VMEM((CH, D), jnp.float32),
            pltpu.VMEM((CH, D), jnp.float32),
            pltpu.VMEM((SV_ROWS, D), jnp.float32),
            pltpu.VMEM_SHARED((96 * 56, D), jnp.float32),
        ] + [pltpu.SemaphoreType.DMA] * 6,
    )
    out = run(idx_packed, smalls, emb_loc_w)
    return out.reshape(SEQ, B, D)
